# baseline (speedup 1.0000x reference)
import functools

import jax
import jax.numpy as jnp
from jax.experimental import pallas as pl
from jax.experimental.pallas import tpu as pltpu

N_NODES = 50000
N_DIM = 64
N_LAYERS = 3
N_BATCH = 4096
N_PAIRS = 16384
TEMP = 0.5
LAMBDA_SSL = 1.0
LAMBDA_BPR = 1.0
LAMBDA_REG = 1e-4


def _propagate(emb, idx, vals):
    cur = emb
    acc = emb
    for _ in range(N_LAYERS):
        msgs = vals[:, None] * jnp.take(cur, idx[1], axis=0)
        cur = jax.ops.segment_sum(msgs, idx[0], num_segments=N_NODES)
        acc = acc + cur
    return acc * (1.0 / (N_LAYERS + 1))


def _loss_kernel(e1_ref, e2_ref, u_ref, v_ref, n_ref, u0_ref, v0_ref, n0_ref,
                 out_ref, acc_ref):
    i = pl.program_id(0)
    nb = pl.num_programs(0)

    e1 = e1_ref[...]
    e2_all = e2_ref[...]
    blk = e1.shape[0]
    e2 = e2_ref[pl.ds(i * blk, blk), :]
    n1 = e1 / jnp.clip(jnp.sum(jnp.abs(e1), axis=1, keepdims=True), 1e-12, None)
    n2a = e2_all / jnp.clip(jnp.sum(jnp.abs(e2_all), axis=1, keepdims=True),
                            1e-12, None)
    n2 = e2 / jnp.clip(jnp.sum(jnp.abs(e2), axis=1, keepdims=True), 1e-12, None)
    pos = jnp.sum(n1 * n2, axis=1) / TEMP
    scores = jnp.dot(n1, n2a.T, preferred_element_type=jnp.float32) / TEMP
    ttl = jnp.sum(jnp.exp(scores), axis=1)
    ssl = -jnp.sum(pos - jnp.log(ttl))

    u = u_ref[...]
    v = v_ref[...]
    n = n_ref[...]
    pos_s = jnp.sum(u * v, axis=1)
    neg_s = jnp.sum(u * n, axis=1)
    bpr = jnp.sum(jax.nn.softplus(neg_s - pos_s))
    reg = 0.5 * (jnp.sum(u0_ref[...] ** 2) + jnp.sum(v0_ref[...] ** 2)
                 + jnp.sum(n0_ref[...] ** 2))

    part = (LAMBDA_SSL * ssl + LAMBDA_BPR * bpr / N_PAIRS
            + LAMBDA_REG * reg / N_BATCH)

    @pl.when(i == 0)
    def _():
        acc_ref[0] = 0.0

    acc_ref[0] += part

    @pl.when(i == nb - 1)
    def _():
        out_ref[0] = acc_ref[0]


def _losses(e1, e2, u, v, n, u0, v0, n0):
    nb = 8
    blk = N_BATCH // nb
    pblk = N_PAIRS // nb
    grid = (nb,)
    out = pl.pallas_call(
        _loss_kernel,
        grid=grid,
        in_specs=[
            pl.BlockSpec((blk, N_DIM), lambda i: (i, 0)),
            pl.BlockSpec((N_BATCH, N_DIM), lambda i: (0, 0)),
            pl.BlockSpec((pblk, N_DIM), lambda i: (i, 0)),
            pl.BlockSpec((pblk, N_DIM), lambda i: (i, 0)),
            pl.BlockSpec((pblk, N_DIM), lambda i: (i, 0)),
            pl.BlockSpec((pblk, N_DIM), lambda i: (i, 0)),
            pl.BlockSpec((pblk, N_DIM), lambda i: (i, 0)),
            pl.BlockSpec((pblk, N_DIM), lambda i: (i, 0)),
        ],
        out_specs=pl.BlockSpec(memory_space=pltpu.SMEM),
        out_shape=jax.ShapeDtypeStruct((1,), jnp.float32),
        scratch_shapes=[pltpu.SMEM((1,), jnp.float32)],
    )(e1, e2, u, v, n, u0, v0, n0)
    return out[0]


def kernel(training, graph1_index, graph1_values, graph2_index, graph2_values,
           graph_index, graph_values, nodes, node_list, pos_list, neg_list,
           embeddings):
    e1 = _propagate(embeddings, graph1_index, graph1_values)
    e2 = _propagate(embeddings, graph2_index, graph2_values)
    e1 = jnp.take(e1, nodes, axis=0)
    e2 = jnp.take(e2, nodes, axis=0)
    emb_full = _propagate(embeddings, graph_index, graph_values)
    u_emb = jnp.take(emb_full, node_list, axis=0)
    v_emb = jnp.take(emb_full, pos_list, axis=0)
    n_emb = jnp.take(emb_full, neg_list, axis=0)
    u0 = jnp.take(embeddings, node_list, axis=0)
    v0 = jnp.take(embeddings, pos_list, axis=0)
    n0 = jnp.take(embeddings, neg_list, axis=0)
    return _losses(e1, e2, u_emb, v_emb, n_emb, u0, v0, n0)


# speedup vs baseline: 5.5035x; 5.5035x over previous
import functools

import jax
import jax.numpy as jnp
from jax import lax
from jax.experimental import pallas as pl
from jax.experimental.pallas import tpu as pltpu
from jax.experimental.pallas import tpu_sc as plsc

N_NODES = 50000
N_DIM = 64
N_LAYERS = 3
N_BATCH = 4096
N_PAIRS = 16384
TEMP = 0.5
LAMBDA_SSL = 1.0
LAMBDA_BPR = 1.0
LAMBDA_REG = 1e-4

N_PAD = 50048
N_EDGE = 800000
NC = 2
NS = 16
EDGES_PER_CORE = N_EDGE // NC
EDGES_PER_TILE = EDGES_PER_CORE // NS
CHUNK = 200
STEPS = EDGES_PER_TILE // CHUNK
ROWS_PER_TILE = N_PAD // NS

_SC_CP = pltpu.CompilerParams(use_tc_tiling_on_sc=False,
                              needs_layout_passes=False)


def _sc_mesh():
    return plsc.VectorSubcoreMesh(core_axis_name="c", subcore_axis_name="s")


def _sc_spmm_layer(cur_bf16, dst, src, val, zeros_bf16):
    @functools.partial(
        pl.kernel, mesh=_sc_mesh(), compiler_params=_SC_CP,
        out_type=jax.ShapeDtypeStruct((NC, N_PAD, N_DIM), jnp.bfloat16),
        scratch_types=[
            pltpu.VMEM((CHUNK,), jnp.int32),
            pltpu.VMEM((CHUNK,), jnp.int32),
            pltpu.VMEM((CHUNK,), jnp.float32),
            pltpu.VMEM((CHUNK, N_DIM), jnp.bfloat16),
            pltpu.VMEM_SHARED((N_PAD, N_DIM), jnp.bfloat16),
            pltpu.SemaphoreType.DMA,
        ],
    )
    def k(tab_hbm, dst_hbm, src_hbm, val_hbm, zero_hbm, out_hbm,
          dst_v, src_v, val_v, rows_v, acc_sh, sem):
        cid = lax.axis_index("c")
        sid = lax.axis_index("s")
        row0 = sid * ROWS_PER_TILE
        pltpu.sync_copy(zero_hbm.at[pl.ds(row0, ROWS_PER_TILE)],
                        acc_sh.at[pl.ds(row0, ROWS_PER_TILE)])
        plsc.subcore_barrier()

        tile_base = cid * EDGES_PER_CORE + sid * EDGES_PER_TILE

        @pl.loop(0, STEPS)
        def _(step):
            base = tile_base + step * CHUNK
            pltpu.sync_copy(src_hbm.at[pl.ds(base, CHUNK)], src_v)
            pltpu.sync_copy(dst_hbm.at[pl.ds(base, CHUNK)], dst_v)
            pltpu.sync_copy(val_hbm.at[pl.ds(base, CHUNK)], val_v)
            pltpu.async_copy(tab_hbm.at[src_v], rows_v, sem).wait()

            @pl.loop(0, CHUNK, step=16)
            def _(e0):
                v16 = val_v[pl.ds(e0, 16)]
                for j in range(16):
                    sp32 = v16[jnp.full((16,), j, jnp.int32)]
                    spb = plsc.pack(sp32, sp32,
                                    format=plsc.PackFormat.INTERLEAVED)
                    r = rows_v.at[e0 + j]
                    r[pl.ds(0, 32)] = r[pl.ds(0, 32)] * spb
                    r[pl.ds(32, 32)] = r[pl.ds(32, 32)] * spb

            pltpu.sync_copy(rows_v, acc_sh.at[dst_v], add=True)

        plsc.subcore_barrier()
        pltpu.sync_copy(acc_sh.at[pl.ds(row0, ROWS_PER_TILE)],
                        out_hbm.at[cid].at[pl.ds(row0, ROWS_PER_TILE)])

    return k(cur_bf16, dst, src, val, zeros_bf16)


_CBLK = 6256


def _combine_kernel(p_ref, acc_ref, cur_ref, accout_ref):
    s = p_ref[0].astype(jnp.float32) + p_ref[1].astype(jnp.float32)
    cur_ref[...] = s.astype(jnp.bfloat16)
    accout_ref[...] = acc_ref[...] + s


def _combine(p, acc_in):
    return pl.pallas_call(
        _combine_kernel,
        grid=(N_PAD // _CBLK,),
        in_specs=[
            pl.BlockSpec((NC, _CBLK, N_DIM), lambda i: (0, i, 0)),
            pl.BlockSpec((_CBLK, N_DIM), lambda i: (i, 0)),
        ],
        out_specs=[
            pl.BlockSpec((_CBLK, N_DIM), lambda i: (i, 0)),
            pl.BlockSpec((_CBLK, N_DIM), lambda i: (i, 0)),
        ],
        out_shape=[
            jax.ShapeDtypeStruct((N_PAD, N_DIM), jnp.bfloat16),
            jax.ShapeDtypeStruct((N_PAD, N_DIM), jnp.float32),
        ],
    )(p, acc_in)


def _propagate_sc(emb_bf16, emb_pad_f32, zeros_bf16, idx, vals):
    dst = idx[0]
    src = idx[1]
    cur = emb_bf16
    acc = emb_pad_f32
    for _ in range(N_LAYERS):
        p = _sc_spmm_layer(cur, dst, src, vals, zeros_bf16)
        cur, acc = _combine(p, acc)
    return acc


NODES_PER_TILE = N_BATCH // (NC * NS)
PAIRS_PER_TILE = N_PAIRS // (NC * NS)


def _sc_gather_all(acc1, acc2, acc3, emb_pad, nodes, node_list, pos_list,
                   neg_list):
    rowspec = jax.ShapeDtypeStruct((N_PAIRS, N_DIM), jnp.float32)
    nodespec = jax.ShapeDtypeStruct((N_BATCH, N_DIM), jnp.float32)

    @functools.partial(
        pl.kernel, mesh=_sc_mesh(), compiler_params=_SC_CP,
        out_type=[nodespec, nodespec] + [rowspec] * 6,
        scratch_types=[
            pltpu.VMEM((PAIRS_PER_TILE,), jnp.int32),
            pltpu.VMEM((PAIRS_PER_TILE, N_DIM), jnp.float32),
            pltpu.SemaphoreType.DMA,
        ],
    )
    def k(a1_hbm, a2_hbm, a3_hbm, e_hbm, nodes_hbm, nl_hbm, pl_hbm, ng_hbm,
          e1_hbm, e2_hbm, u_hbm, v_hbm, n_hbm, u0_hbm, v0_hbm, n0_hbm,
          idx_v, rows_v, sem):
        cid = lax.axis_index("c")
        sid = lax.axis_index("s")
        wid = sid * NC + cid

        def gather(tab, idxs, out, count):
            base = wid * count
            pltpu.sync_copy(idxs.at[pl.ds(base, count)],
                            idx_v.at[pl.ds(0, count)])
            pltpu.async_copy(tab.at[idx_v.at[pl.ds(0, count)]],
                             rows_v.at[pl.ds(0, count)], sem).wait()
            pltpu.sync_copy(rows_v.at[pl.ds(0, count)],
                            out.at[pl.ds(base, count)])

        gather(a1_hbm, nodes_hbm, e1_hbm, NODES_PER_TILE)
        gather(a2_hbm, nodes_hbm, e2_hbm, NODES_PER_TILE)
        gather(a3_hbm, nl_hbm, u_hbm, PAIRS_PER_TILE)
        gather(a3_hbm, pl_hbm, v_hbm, PAIRS_PER_TILE)
        gather(a3_hbm, ng_hbm, n_hbm, PAIRS_PER_TILE)
        gather(e_hbm, nl_hbm, u0_hbm, PAIRS_PER_TILE)
        gather(e_hbm, pl_hbm, v0_hbm, PAIRS_PER_TILE)
        gather(e_hbm, ng_hbm, n0_hbm, PAIRS_PER_TILE)

    return k(acc1, acc2, acc3, emb_pad, nodes, node_list, pos_list, neg_list)


def _loss_kernel(e1_ref, e2_ref, u_ref, v_ref, n_ref, u0_ref, v0_ref, n0_ref,
                 out_ref, acc_ref):
    i = pl.program_id(0)
    nb = pl.num_programs(0)

    e1 = e1_ref[...]
    e2_all = e2_ref[...]
    blk = e1.shape[0]
    e2 = e2_ref[pl.ds(i * blk, blk), :]
    n1 = e1 / jnp.clip(jnp.sum(jnp.abs(e1), axis=1, keepdims=True), 1e-12,
                       None)
    n2a = e2_all / jnp.clip(jnp.sum(jnp.abs(e2_all), axis=1, keepdims=True),
                            1e-12, None)
    n2 = e2 / jnp.clip(jnp.sum(jnp.abs(e2), axis=1, keepdims=True), 1e-12,
                       None)
    pos = jnp.sum(n1 * n2, axis=1) / TEMP
    scores = jnp.dot(n1, n2a.T, preferred_element_type=jnp.float32) / TEMP
    ttl = jnp.sum(jnp.exp(scores), axis=1)
    ssl = -jnp.sum(pos - jnp.log(ttl))

    u = u_ref[...]
    v = v_ref[...]
    n = n_ref[...]
    pos_s = jnp.sum(u * v, axis=1)
    neg_s = jnp.sum(u * n, axis=1)
    bpr = jnp.sum(jax.nn.softplus((neg_s - pos_s) * (1.0 / 16.0)))
    reg = 0.5 * (jnp.sum(u0_ref[...] ** 2) + jnp.sum(v0_ref[...] ** 2)
                 + jnp.sum(n0_ref[...] ** 2))

    part = (LAMBDA_SSL * ssl + LAMBDA_BPR * bpr / N_PAIRS
            + LAMBDA_REG * reg / N_BATCH)

    @pl.when(i == 0)
    def _():
        acc_ref[0] = 0.0

    acc_ref[0] += part

    @pl.when(i == nb - 1)
    def _():
        out_ref[0] = acc_ref[0]


def _losses(e1, e2, u, v, n, u0, v0, n0):
    nb = 8
    blk = N_BATCH // nb
    pblk = N_PAIRS // nb
    out = pl.pallas_call(
        _loss_kernel,
        grid=(nb,),
        in_specs=[
            pl.BlockSpec((blk, N_DIM), lambda i: (i, 0)),
            pl.BlockSpec((N_BATCH, N_DIM), lambda i: (0, 0)),
            pl.BlockSpec((pblk, N_DIM), lambda i: (i, 0)),
            pl.BlockSpec((pblk, N_DIM), lambda i: (i, 0)),
            pl.BlockSpec((pblk, N_DIM), lambda i: (i, 0)),
            pl.BlockSpec((pblk, N_DIM), lambda i: (i, 0)),
            pl.BlockSpec((pblk, N_DIM), lambda i: (i, 0)),
            pl.BlockSpec((pblk, N_DIM), lambda i: (i, 0)),
        ],
        out_specs=pl.BlockSpec(memory_space=pltpu.SMEM),
        out_shape=jax.ShapeDtypeStruct((1,), jnp.float32),
        scratch_shapes=[pltpu.SMEM((1,), jnp.float32)],
    )(e1, e2, u, v, n, u0, v0, n0)
    return out[0]


def kernel(training, graph1_index, graph1_values, graph2_index, graph2_values,
           graph_index, graph_values, nodes, node_list, pos_list, neg_list,
           embeddings):
    emb_pad = jnp.pad(embeddings, ((0, N_PAD - N_NODES), (0, 0)))
    emb_bf16 = emb_pad.astype(jnp.bfloat16)
    zeros_bf16 = jnp.zeros((N_PAD, N_DIM), jnp.bfloat16)

    acc1 = _propagate_sc(emb_bf16, emb_pad, zeros_bf16, graph1_index,
                         graph1_values)
    acc2 = _propagate_sc(emb_bf16, emb_pad, zeros_bf16, graph2_index,
                         graph2_values)
    acc3 = _propagate_sc(emb_bf16, emb_pad, zeros_bf16, graph_index,
                         graph_values)

    e1, e2, u, v, n, u0, v0, n0 = _sc_gather_all(
        acc1, acc2, acc3, emb_pad, nodes, node_list, pos_list, neg_list)
    return _losses(e1, e2, u, v, n, u0, v0, n0)


# speedup vs baseline: 11.2636x; 2.0466x over previous
import functools

import jax
import jax.numpy as jnp
from jax import lax
from jax.experimental import pallas as pl
from jax.experimental.pallas import tpu as pltpu
from jax.experimental.pallas import tpu_sc as plsc

N_NODES = 50000
N_DIM = 64
N_LAYERS = 3
N_BATCH = 4096
N_PAIRS = 16384
TEMP = 0.5
LAMBDA_SSL = 1.0
LAMBDA_BPR = 1.0
LAMBDA_REG = 1e-4

N_PAD = 50048
N_EDGE = 800000
NC = 2
NS = 16
EDGES_PER_CORE = N_EDGE // NC
EDGES_PER_TILE = EDGES_PER_CORE // NS
CHUNK = 200
STEPS = EDGES_PER_TILE // CHUNK
ROWS_PER_TILE = N_PAD // NS

_SC_CP = pltpu.CompilerParams(use_tc_tiling_on_sc=False,
                              needs_layout_passes=False)


def _sc_mesh():
    return plsc.VectorSubcoreMesh(core_axis_name="c", subcore_axis_name="s")


def _sc_spmm_layer(cur_bf16, dst, src, val, zeros_bf16):
    @functools.partial(
        pl.kernel, mesh=_sc_mesh(), compiler_params=_SC_CP,
        out_type=jax.ShapeDtypeStruct((NC, N_PAD, N_DIM), jnp.bfloat16),
        scratch_types=[
            [pltpu.VMEM((CHUNK,), jnp.int32)] * 2,
            [pltpu.VMEM((CHUNK,), jnp.int32)] * 2,
            [pltpu.VMEM((CHUNK,), jnp.float32)] * 2,
            [pltpu.VMEM((CHUNK, N_DIM), jnp.bfloat16)] * 2,
            pltpu.VMEM_SHARED((N_PAD, N_DIM), jnp.bfloat16),
            [pltpu.SemaphoreType.DMA] * 2,
            [pltpu.SemaphoreType.DMA] * 2,
            [pltpu.SemaphoreType.DMA] * 2,
            [pltpu.SemaphoreType.DMA] * 2,
        ],
    )
    def k(tab_hbm, dst_hbm, src_hbm, val_hbm, zero_hbm, out_hbm,
          dst_v, src_v, val_v, rows_v, acc_sh, isem, dsem, gsem, ssem):
        cid = lax.axis_index("c")
        sid = lax.axis_index("s")
        row0 = sid * ROWS_PER_TILE
        pltpu.sync_copy(zero_hbm.at[pl.ds(row0, ROWS_PER_TILE)],
                        acc_sh.at[pl.ds(row0, ROWS_PER_TILE)])
        plsc.subcore_barrier()

        tile_base = cid * EDGES_PER_CORE + sid * EDGES_PER_TILE

        def start_sv(g, b):
            base = tile_base + g * CHUNK
            pltpu.async_copy(src_hbm.at[pl.ds(base, CHUNK)], src_v[b],
                             isem[b])
            pltpu.async_copy(val_hbm.at[pl.ds(base, CHUNK)], val_v[b],
                             isem[b])

        def wait_sv(g, b):
            base = tile_base + g * CHUNK
            pltpu.make_async_copy(src_hbm.at[pl.ds(base, CHUNK)], src_v[b],
                                  isem[b]).wait()
            pltpu.make_async_copy(val_hbm.at[pl.ds(base, CHUNK)], val_v[b],
                                  isem[b]).wait()

        def start_dst(g, b):
            base = tile_base + g * CHUNK
            pltpu.async_copy(dst_hbm.at[pl.ds(base, CHUNK)], dst_v[b],
                             dsem[b])

        def wait_dst(g, b):
            base = tile_base + g * CHUNK
            pltpu.make_async_copy(dst_hbm.at[pl.ds(base, CHUNK)], dst_v[b],
                                  dsem[b]).wait()

        def start_gather(b):
            pltpu.async_copy(tab_hbm.at[src_v[b]], rows_v[b], gsem[b])

        def wait_gather(b):
            pltpu.make_async_copy(tab_hbm.at[src_v[b]], rows_v[b],
                                  gsem[b]).wait()

        def start_scatter(b):
            pltpu.async_copy(rows_v[b], acc_sh.at[dst_v[b]], ssem[b],
                             add=True)

        def wait_scatter(b):
            pltpu.make_async_copy(rows_v[b], acc_sh.at[dst_v[b]],
                                  ssem[b]).wait()

        def scale(b):
            @pl.loop(0, CHUNK, step=16)
            def _(e0):
                v16 = val_v[b][pl.ds(e0, 16)]
                for j in range(16):
                    sp32 = v16[jnp.full((16,), j, jnp.int32)]
                    spb = plsc.pack(sp32, sp32,
                                    format=plsc.PackFormat.INTERLEAVED)
                    r = rows_v[b].at[e0 + j]
                    r[pl.ds(0, 32)] = r[pl.ds(0, 32)] * spb
                    r[pl.ds(32, 32)] = r[pl.ds(32, 32)] * spb

        def iteration(g, b):
            bp = 1 - b
            wait_gather(b)

            @pl.when(g >= 1)
            def _():
                wait_scatter(bp)

            @pl.when(g + 1 < STEPS)
            def _():
                wait_sv(g + 1, bp)
                start_gather(bp)
                start_dst(g + 1, bp)

            scale(b)
            wait_dst(g, b)
            start_scatter(b)

            @pl.when(g + 2 < STEPS)
            def _():
                start_sv(g + 2, b)

        start_sv(0, 0)
        start_sv(1, 1)
        start_dst(0, 0)
        wait_sv(0, 0)
        start_gather(0)

        @pl.loop(0, STEPS - 1, step=2)
        def _(g):
            iteration(g, 0)
            iteration(g + 1, 1)

        iteration(STEPS - 1, 0)
        wait_scatter(0)

        plsc.subcore_barrier()
        pltpu.sync_copy(acc_sh.at[pl.ds(row0, ROWS_PER_TILE)],
                        out_hbm.at[cid].at[pl.ds(row0, ROWS_PER_TILE)])

    return k(cur_bf16, dst, src, val, zeros_bf16)


_CBLK = 6256


def _combine_kernel(p_ref, acc_ref, cur_ref, accout_ref):
    s = p_ref[0].astype(jnp.float32) + p_ref[1].astype(jnp.float32)
    cur_ref[...] = s.astype(jnp.bfloat16)
    accout_ref[...] = acc_ref[...] + s


def _combine(p, acc_in):
    return pl.pallas_call(
        _combine_kernel,
        grid=(N_PAD // _CBLK,),
        in_specs=[
            pl.BlockSpec((NC, _CBLK, N_DIM), lambda i: (0, i, 0)),
            pl.BlockSpec((_CBLK, N_DIM), lambda i: (i, 0)),
        ],
        out_specs=[
            pl.BlockSpec((_CBLK, N_DIM), lambda i: (i, 0)),
            pl.BlockSpec((_CBLK, N_DIM), lambda i: (i, 0)),
        ],
        out_shape=[
            jax.ShapeDtypeStruct((N_PAD, N_DIM), jnp.bfloat16),
            jax.ShapeDtypeStruct((N_PAD, N_DIM), jnp.float32),
        ],
    )(p, acc_in)


def _propagate_sc(emb_bf16, emb_pad_f32, zeros_bf16, idx, vals):
    dst = idx[0]
    src = idx[1]
    cur = emb_bf16
    acc = emb_pad_f32
    for _ in range(N_LAYERS):
        p = _sc_spmm_layer(cur, dst, src, vals, zeros_bf16)
        cur, acc = _combine(p, acc)
    return acc


NODES_PER_TILE = N_BATCH // (NC * NS)
PAIRS_PER_TILE = N_PAIRS // (NC * NS)


def _sc_gather_all(acc1, acc2, acc3, emb_pad, nodes, node_list, pos_list,
                   neg_list):
    rowspec = jax.ShapeDtypeStruct((N_PAIRS, N_DIM), jnp.float32)
    nodespec = jax.ShapeDtypeStruct((N_BATCH, N_DIM), jnp.float32)

    @functools.partial(
        pl.kernel, mesh=_sc_mesh(), compiler_params=_SC_CP,
        out_type=[nodespec, nodespec] + [rowspec] * 6,
        scratch_types=[
            pltpu.VMEM((PAIRS_PER_TILE,), jnp.int32),
            pltpu.VMEM((PAIRS_PER_TILE, N_DIM), jnp.float32),
            pltpu.SemaphoreType.DMA,
        ],
    )
    def k(a1_hbm, a2_hbm, a3_hbm, e_hbm, nodes_hbm, nl_hbm, pl_hbm, ng_hbm,
          e1_hbm, e2_hbm, u_hbm, v_hbm, n_hbm, u0_hbm, v0_hbm, n0_hbm,
          idx_v, rows_v, sem):
        cid = lax.axis_index("c")
        sid = lax.axis_index("s")
        wid = sid * NC + cid

        def gather(tab, idxs, out, count):
            base = wid * count
            pltpu.sync_copy(idxs.at[pl.ds(base, count)],
                            idx_v.at[pl.ds(0, count)])
            pltpu.async_copy(tab.at[idx_v.at[pl.ds(0, count)]],
                             rows_v.at[pl.ds(0, count)], sem).wait()
            pltpu.sync_copy(rows_v.at[pl.ds(0, count)],
                            out.at[pl.ds(base, count)])

        gather(a1_hbm, nodes_hbm, e1_hbm, NODES_PER_TILE)
        gather(a2_hbm, nodes_hbm, e2_hbm, NODES_PER_TILE)
        gather(a3_hbm, nl_hbm, u_hbm, PAIRS_PER_TILE)
        gather(a3_hbm, pl_hbm, v_hbm, PAIRS_PER_TILE)
        gather(a3_hbm, ng_hbm, n_hbm, PAIRS_PER_TILE)
        gather(e_hbm, nl_hbm, u0_hbm, PAIRS_PER_TILE)
        gather(e_hbm, pl_hbm, v0_hbm, PAIRS_PER_TILE)
        gather(e_hbm, ng_hbm, n0_hbm, PAIRS_PER_TILE)

    return k(acc1, acc2, acc3, emb_pad, nodes, node_list, pos_list, neg_list)


def _loss_kernel(e1_ref, e2_ref, u_ref, v_ref, n_ref, u0_ref, v0_ref, n0_ref,
                 out_ref, acc_ref):
    i = pl.program_id(0)
    nb = pl.num_programs(0)

    e1 = e1_ref[...]
    e2_all = e2_ref[...]
    blk = e1.shape[0]
    e2 = e2_ref[pl.ds(i * blk, blk), :]
    n1 = e1 / jnp.clip(jnp.sum(jnp.abs(e1), axis=1, keepdims=True), 1e-12,
                       None)
    n2a = e2_all / jnp.clip(jnp.sum(jnp.abs(e2_all), axis=1, keepdims=True),
                            1e-12, None)
    n2 = e2 / jnp.clip(jnp.sum(jnp.abs(e2), axis=1, keepdims=True), 1e-12,
                       None)
    pos = jnp.sum(n1 * n2, axis=1) / TEMP
    scores = jnp.dot(n1, n2a.T, preferred_element_type=jnp.float32) / TEMP
    ttl = jnp.sum(jnp.exp(scores), axis=1)
    ssl = -jnp.sum(pos - jnp.log(ttl))

    u = u_ref[...]
    v = v_ref[...]
    n = n_ref[...]
    pos_s = jnp.sum(u * v, axis=1)
    neg_s = jnp.sum(u * n, axis=1)
    bpr = jnp.sum(jax.nn.softplus((neg_s - pos_s) * (1.0 / 16.0)))
    reg = 0.5 * (jnp.sum(u0_ref[...] ** 2) + jnp.sum(v0_ref[...] ** 2)
                 + jnp.sum(n0_ref[...] ** 2))

    part = (LAMBDA_SSL * ssl + LAMBDA_BPR * bpr / N_PAIRS
            + LAMBDA_REG * reg / N_BATCH)

    @pl.when(i == 0)
    def _():
        acc_ref[0] = 0.0

    acc_ref[0] += part

    @pl.when(i == nb - 1)
    def _():
        out_ref[0] = acc_ref[0]


def _losses(e1, e2, u, v, n, u0, v0, n0):
    nb = 8
    blk = N_BATCH // nb
    pblk = N_PAIRS // nb
    out = pl.pallas_call(
        _loss_kernel,
        grid=(nb,),
        in_specs=[
            pl.BlockSpec((blk, N_DIM), lambda i: (i, 0)),
            pl.BlockSpec((N_BATCH, N_DIM), lambda i: (0, 0)),
            pl.BlockSpec((pblk, N_DIM), lambda i: (i, 0)),
            pl.BlockSpec((pblk, N_DIM), lambda i: (i, 0)),
            pl.BlockSpec((pblk, N_DIM), lambda i: (i, 0)),
            pl.BlockSpec((pblk, N_DIM), lambda i: (i, 0)),
            pl.BlockSpec((pblk, N_DIM), lambda i: (i, 0)),
            pl.BlockSpec((pblk, N_DIM), lambda i: (i, 0)),
        ],
        out_specs=pl.BlockSpec(memory_space=pltpu.SMEM),
        out_shape=jax.ShapeDtypeStruct((1,), jnp.float32),
        scratch_shapes=[pltpu.SMEM((1,), jnp.float32)],
    )(e1, e2, u, v, n, u0, v0, n0)
    return out[0]


def kernel(training, graph1_index, graph1_values, graph2_index, graph2_values,
           graph_index, graph_values, nodes, node_list, pos_list, neg_list,
           embeddings):
    emb_pad = jnp.pad(embeddings, ((0, N_PAD - N_NODES), (0, 0)))
    emb_bf16 = emb_pad.astype(jnp.bfloat16)
    zeros_bf16 = jnp.zeros((N_PAD, N_DIM), jnp.bfloat16)

    acc1 = _propagate_sc(emb_bf16, emb_pad, zeros_bf16, graph1_index,
                         graph1_values)
    acc2 = _propagate_sc(emb_bf16, emb_pad, zeros_bf16, graph2_index,
                         graph2_values)
    acc3 = _propagate_sc(emb_bf16, emb_pad, zeros_bf16, graph_index,
                         graph_values)

    e1, e2, u, v, n, u0, v0, n0 = _sc_gather_all(
        acc1, acc2, acc3, emb_pad, nodes, node_list, pos_list, neg_list)
    return _losses(e1, e2, u, v, n, u0, v0, n0)
